# Initial kernel scaffold; baseline (speedup 1.0000x reference)
import jax
import jax.numpy as jnp
from jax.experimental import pallas as pl


def kernel(x, edge_index, edge_attr, W_enc, b_enc, W_edge, b_edge, W_g0, b_g0, W_g1, b_g1, W_g2, b_g2, W_o1, b_o1, W_o2, b_o2):
    raise NotImplementedError("write your pallas kernel here")


# speedup vs baseline: 17.1049x; 17.1049x over previous
import functools

import jax
import jax.numpy as jnp
from jax import lax
from jax.experimental import pallas as pl
from jax.experimental.pallas import tpu as pltpu
from jax.experimental.pallas import tpu_sc as plsc

N = 10000
H = 64
HH = H // 2
NC = 2
NS = 16
NW = NC * NS
CHUNK = 128
CPW = 160
EPAD = NS * CPW * CHUNK
NPAD = 10112
DUMMY = N
RPW = NPAD // NS

_mesh = plsc.VectorSubcoreMesh(core_axis_name="c", subcore_axis_name="s")
_sc_params = pltpu.CompilerParams(use_tc_tiling_on_sc=False,
                                 needs_layout_passes=False)
_f32 = jnp.float32


@functools.partial(
    pl.kernel,
    out_type=jax.ShapeDtypeStruct((NW, NPAD), _f32),
    mesh=_mesh,
    compiler_params=_sc_params,
    scratch_types=[
        pltpu.VMEM((CPW, CHUNK), jnp.int32),
        pltpu.VMEM((NPAD,), _f32),
    ],
)
def _deg_sc(dst_hbm, zeros_hbm, out_hbm, dst_v, hist):
    c = lax.axis_index("c")
    s = lax.axis_index("s")
    wid = s * NC + c

    pltpu.sync_copy(zeros_hbm, hist)
    pltpu.sync_copy(dst_hbm.at[s], dst_v)
    ones_v = jnp.ones((16,), _f32)

    def outer(j, carry):
        for k in range(CHUNK // 16):
            idx = dst_v[j, pl.ds(16 * k, 16)]
            plsc.addupdate_scatter(hist, [idx], ones_v)
        return carry

    lax.fori_loop(0, CPW, outer, 0)
    pltpu.sync_copy(hist, out_hbm.at[wid])


@functools.partial(
    pl.kernel,
    out_type=jax.ShapeDtypeStruct((NC, NPAD, HH), _f32),
    mesh=_mesh,
    compiler_params=_sc_params,
    scratch_types=[
        pltpu.VMEM((CPW, CHUNK), jnp.int32),
        pltpu.VMEM((CPW, CHUNK), jnp.int32),
        pltpu.VMEM((4, CHUNK, HH), _f32),
        pltpu.VMEM((RPW, HH), _f32),
        pltpu.VMEM_SHARED((NPAD, HH), _f32),
        pltpu.SemaphoreType.DMA,
        pltpu.SemaphoreType.DMA,
        pltpu.SemaphoreType.DMA,
        pltpu.SemaphoreType.DMA,
    ],
)
def _agg_sc(hs_hbm, src_hbm, dst_hbm, zeros_hbm, out_hbm,
            src_v, dst_v, rb, zb, acc, g0, g1, g2, g3):
    c = lax.axis_index("c")
    s = lax.axis_index("s")
    r0 = s * RPW
    gsem = [g0, g1, g2, g3]
    table = hs_hbm.at[c]

    pltpu.sync_copy(zeros_hbm.at[pl.ds(r0, RPW)], zb)
    pltpu.sync_copy(zb, acc.at[pl.ds(r0, RPW)])
    pltpu.sync_copy(src_hbm.at[s], src_v)
    pltpu.sync_copy(dst_hbm.at[s], dst_v)
    plsc.subcore_barrier()

    pltpu.async_copy(table.at[src_v.at[0]], rb.at[0], g0)
    pltpu.async_copy(table.at[src_v.at[1]], rb.at[1], g1)

    def outer(t, carry):
        for b in range(4):
            j = 4 * t + b
            pltpu.make_async_copy(
                table.at[src_v.at[j]], rb.at[b], gsem[b]).wait()
            pltpu.sync_copy(rb.at[b], acc.at[dst_v.at[j]], add=True)
            b2 = (b + 2) % 4

            @pl.when(j + 2 < CPW)
            def _():
                pltpu.async_copy(
                    table.at[src_v.at[j + 2]], rb.at[b2], gsem[b2])
        return carry

    lax.fori_loop(0, CPW // 4, outer, 0)
    plsc.subcore_barrier()
    pltpu.sync_copy(acc.at[pl.ds(r0, RPW)], zb)
    pltpu.sync_copy(zb, out_hbm.at[c].at[pl.ds(r0, RPW)])


def _enc_tc_body(x_ref, we_ref, be_ref, cnt_ref, h_ref, hs_ref, dinv_ref):
    deg = 1.0 + 0.5 * jnp.sum(cnt_ref[:, :N], axis=0)[:, None]
    dinv = lax.rsqrt(deg)
    h = jnp.dot(x_ref[...], we_ref[...],
                preferred_element_type=_f32) + be_ref[...]
    h_ref[...] = h
    hs = h * dinv
    hs_ref[0] = hs[:, :HH]
    hs_ref[1] = hs[:, HH:]
    dinv_ref[...] = dinv


def _mid_tc_body(agg_ref, h_ref, dinv_ref, w_ref, b_ref, h_o, hs_o):
    dinv = dinv_ref[...]
    ssum = jnp.concatenate([agg_ref[0, :N, :], agg_ref[1, :N, :]], axis=1)
    m = ssum * dinv + h_ref[...] * (dinv * dinv)
    h = jnp.maximum(
        jnp.dot(m, w_ref[...], preferred_element_type=_f32) + b_ref[...],
        0.0)
    h_o[...] = h
    hs = h * dinv
    hs_o[0] = hs[:, :HH]
    hs_o[1] = hs[:, HH:]


def _fin_tc_body(h_ref, wo1_ref, bo1_ref, wo2_ref, bo2_ref, out_ref):
    g = jnp.mean(h_ref[...], axis=0, keepdims=True)
    hid = jnp.maximum(
        jnp.dot(g, wo1_ref[...], preferred_element_type=_f32)
        + bo1_ref[...], 0.0)
    out_ref[...] = (
        jnp.dot(hid, wo2_ref[...], preferred_element_type=_f32)
        + bo2_ref[...])


_enc_tc = pl.pallas_call(
    _enc_tc_body,
    out_shape=[
        jax.ShapeDtypeStruct((N, H), _f32),
        jax.ShapeDtypeStruct((NC, N, HH), _f32),
        jax.ShapeDtypeStruct((N, 1), _f32),
    ],
)

_mid_tc = pl.pallas_call(
    _mid_tc_body,
    out_shape=[
        jax.ShapeDtypeStruct((N, H), _f32),
        jax.ShapeDtypeStruct((NC, N, HH), _f32),
    ],
)

_fin_tc = pl.pallas_call(
    _fin_tc_body,
    out_shape=jax.ShapeDtypeStruct((1, 1), _f32),
)


def kernel(x, edge_index, edge_attr, W_enc, b_enc, W_edge, b_edge,
           W_g0, b_g0, W_g1, b_g1, W_g2, b_g2, W_o1, b_o1, W_o2, b_o2):
    del edge_attr, W_edge, b_edge
    src = edge_index[0].astype(jnp.int32)
    dst = edge_index[1].astype(jnp.int32)
    pad = EPAD - src.shape[0]
    src_p = jnp.concatenate(
        [src, jnp.zeros((pad,), jnp.int32)]).reshape(NS, CPW, CHUNK)
    dst_p = jnp.concatenate(
        [dst, jnp.full((pad,), DUMMY, jnp.int32)]).reshape(NS, CPW, CHUNK)

    zeros1 = jnp.zeros((NPAD,), _f32)
    zeros32 = jnp.zeros((NPAD, HH), _f32)

    cnt = _deg_sc(dst_p, zeros1)
    h, hs, dinv = _enc_tc(x, W_enc, b_enc.reshape(1, H), cnt)

    Ws = jnp.stack([W_g0, W_g1, W_g2])
    bs = jnp.stack([b_g0.reshape(1, H), b_g1.reshape(1, H),
                    b_g2.reshape(1, H)])

    def layer(l, carry):
        h_l, hs_l = carry
        agg_l = _agg_sc(hs_l, src_p, dst_p, zeros32)
        return tuple(_mid_tc(agg_l, h_l, dinv, Ws[l], bs[l]))

    h, hs = lax.fori_loop(0, 3, layer, (h, hs))

    return _fin_tc(h, W_o1, b_o1.reshape(1, H // 2), W_o2, b_o2.reshape(1, 1))
